# baseline (speedup 1.0000x reference)
import jax
import jax.numpy as jnp
from jax.experimental import pallas as pl

CONNECTIVITY_RADIUS = 0.5
MAX_NUM_NEIGHBORS = 128
N = 8192
HALF = N // 2
R = 128


def _lex_le(av, ai, bv, bi):
    return (av < bv) | ((av == bv) & (ai < bi))


def _ce(av, ai, bv, bi, asc):
    a_first = _lex_le(av, ai, bv, bi)
    take_a = a_first == asc
    fv = jnp.where(take_a, av, bv)
    sv = jnp.where(take_a, bv, av)
    fi = jnp.where(take_a, ai, bi)
    si = jnp.where(take_a, bi, ai)
    return fv, fi, sv, si


def _sort_chunks_128(val, idx):
    n = val.shape[0]
    k = 2
    while k <= 128:
        j = k // 2
        while j >= 1:
            g = n // (2 * j)
            v = val.reshape(g, 2, j, R)
            x = idx.reshape(g, 2, j, R)
            gids = jax.lax.broadcasted_iota(jnp.int32, (g, j, R), 0)
            asc = (((gids * (2 * j)) % 128) & k) == 0
            fv, fi, sv, si = _ce(v[:, 0], x[:, 0], v[:, 1], x[:, 1], asc)
            val = jnp.concatenate([fv[:, None], sv[:, None]], axis=1)
            val = val.reshape(n, R)
            idx = jnp.concatenate([fi[:, None], si[:, None]], axis=1)
            idx = idx.reshape(n, R)
            j //= 2
        k *= 2
    return val, idx


def _rev128(x):
    m = x.shape[0]
    j = 64
    while j >= 1:
        y = x.reshape(m, 128 // (2 * j), 2, j, R)
        x = jnp.concatenate([y[:, :, 1:2], y[:, :, 0:1]], axis=2)
        x = x.reshape(m, 128, R)
        j //= 2
    return x


def _merge_halve(val, idx):
    m = val.shape[0] // 256
    v = val.reshape(m, 2, 128, R)
    x = idx.reshape(m, 2, 128, R)
    av, ai = v[:, 0], x[:, 0]
    bv, bi = _rev128(v[:, 1]), _rev128(x[:, 1])
    a_first = _lex_le(av, ai, bv, bi)
    val = jnp.where(a_first, av, bv)
    idx = jnp.where(a_first, ai, bi)
    j = 64
    while j >= 1:
        g = 128 // (2 * j)
        v = val.reshape(m, g, 2, j, R)
        x = idx.reshape(m, g, 2, j, R)
        fv, fi, sv, si = _ce(v[:, :, 0], x[:, :, 0], v[:, :, 1], x[:, :, 1], True)
        val = jnp.concatenate([fv[:, :, None], sv[:, :, None]], axis=2)
        val = val.reshape(m, 128, R)
        idx = jnp.concatenate([fi[:, :, None], si[:, :, None]], axis=2)
        idx = idx.reshape(m, 128, R)
        j //= 2
    return val.reshape(m * 128, R), idx.reshape(m * 128, R)


def _topk_body(p_rows_ref, p_cols_ref, out_ref):
    pr = p_rows_ref[...]
    pc = p_cols_ref[...]
    sq_r = jnp.sum(pr * pr, axis=1)
    sq_c = jnp.sum(pc * pc, axis=1)
    cross = jax.lax.dot_general(pc, pr, (((1,), (1,)), ((), ())),
                                preferred_element_type=jnp.float32)
    d2 = sq_c[:, None] + sq_r[None, :] - 2.0 * cross
    d2 = jnp.maximum(d2, 0.0)

    i = pl.program_id(0)
    row_ids = i * R + jax.lax.broadcasted_iota(jnp.int32, d2.shape, 1)
    col_base = (i * R) // HALF * HALF
    cols = jax.lax.broadcasted_iota(jnp.int32, d2.shape, 0)
    r2 = jnp.float32(CONNECTIVITY_RADIUS * CONNECTIVITY_RADIUS)
    valid = (row_ids != (col_base + cols)) & (d2 <= r2)
    val = jnp.where(valid, d2, jnp.inf)
    idx = cols

    val, idx = _sort_chunks_128(val, idx)
    while val.shape[0] > 128:
        val, idx = _merge_halve(val, idx)

    out_ref[...] = jnp.where(val < jnp.inf, idx + col_base, -1)


def kernel(particle_locations, num_particles_per_example):
    del num_particles_per_example
    senders_t = pl.pallas_call(
        _topk_body,
        grid=(N // R,),
        in_specs=[
            pl.BlockSpec((R, 2), lambda i: (i, 0)),
            pl.BlockSpec((HALF, 2), lambda i: ((i * R) // HALF, 0)),
        ],
        out_specs=pl.BlockSpec((MAX_NUM_NEIGHBORS, R), lambda i: (0, i)),
        out_shape=jax.ShapeDtypeStruct((MAX_NUM_NEIGHBORS, N), jnp.int32),
    )(particle_locations, particle_locations)
    senders = senders_t.T
    receivers = jnp.where(senders >= 0, jnp.arange(N, dtype=jnp.int32)[:, None], -1)
    return receivers.reshape(-1), senders.reshape(-1)


# speedup vs baseline: 1.9886x; 1.9886x over previous
import jax
import jax.numpy as jnp
from jax.experimental import pallas as pl

CONNECTIVITY_RADIUS = 0.5
MAX_NUM_NEIGHBORS = 128
N = 8192
HALF = N // 2
R = 256


def _lex_le(av, ai, bv, bi):
    return (av < bv) | ((av == bv) & (ai < bi))


def _ce(av, ai, bv, bi, asc):
    a_first = _lex_le(av, ai, bv, bi)
    take_a = a_first == asc
    fv = jnp.where(take_a, av, bv)
    sv = jnp.where(take_a, bv, av)
    fi = jnp.where(take_a, ai, bi)
    si = jnp.where(take_a, bi, ai)
    return fv, fi, sv, si


def _sort_chunks_128(val, idx):
    n = val.shape[0]
    k = 2
    while k <= 128:
        j = k // 2
        while j >= 1:
            g = n // (2 * j)
            v = val.reshape(g, 2, j, R)
            x = idx.reshape(g, 2, j, R)
            gids = jax.lax.broadcasted_iota(jnp.int32, (g, j, R), 0)
            asc = (((gids * (2 * j)) % 128) & k) == 0
            fv, fi, sv, si = _ce(v[:, 0], x[:, 0], v[:, 1], x[:, 1], asc)
            val = jnp.concatenate([fv[:, None], sv[:, None]], axis=1)
            val = val.reshape(n, R)
            idx = jnp.concatenate([fi[:, None], si[:, None]], axis=1)
            idx = idx.reshape(n, R)
            j //= 2
        k *= 2
    return val, idx


def _rev128(x):
    m = x.shape[0]
    j = 64
    while j >= 1:
        y = x.reshape(m, 128 // (2 * j), 2, j, R)
        x = jnp.concatenate([y[:, :, 1:2], y[:, :, 0:1]], axis=2)
        x = x.reshape(m, 128, R)
        j //= 2
    return x


def _merge_halve(val, idx):
    m = val.shape[0] // 256
    v = val.reshape(m, 2, 128, R)
    x = idx.reshape(m, 2, 128, R)
    av, ai = v[:, 0], x[:, 0]
    bv, bi = _rev128(v[:, 1]), _rev128(x[:, 1])
    a_first = _lex_le(av, ai, bv, bi)
    val = jnp.where(a_first, av, bv)
    idx = jnp.where(a_first, ai, bi)
    j = 64
    while j >= 1:
        g = 128 // (2 * j)
        v = val.reshape(m, g, 2, j, R)
        x = idx.reshape(m, g, 2, j, R)
        fv, fi, sv, si = _ce(v[:, :, 0], x[:, :, 0], v[:, :, 1], x[:, :, 1], True)
        val = jnp.concatenate([fv[:, :, None], sv[:, :, None]], axis=2)
        val = val.reshape(m, 128, R)
        idx = jnp.concatenate([fi[:, :, None], si[:, :, None]], axis=2)
        idx = idx.reshape(m, 128, R)
        j //= 2
    return val.reshape(m * 128, R), idx.reshape(m * 128, R)


def _topk_body(p_rows_ref, p_cols_ref, out_ref):
    pr = p_rows_ref[...]
    pc = p_cols_ref[...]
    sq_r = jnp.sum(pr * pr, axis=1)
    sq_c = jnp.sum(pc * pc, axis=1)
    cross = jax.lax.dot_general(pc, pr, (((1,), (1,)), ((), ())),
                                preferred_element_type=jnp.float32)
    d2 = sq_c[:, None] + sq_r[None, :] - 2.0 * cross
    d2 = jnp.maximum(d2, 0.0)

    i = pl.program_id(0)
    row_ids = i * R + jax.lax.broadcasted_iota(jnp.int32, d2.shape, 1)
    col_base = (i * R) // HALF * HALF
    cols = jax.lax.broadcasted_iota(jnp.int32, d2.shape, 0)
    r2 = jnp.float32(CONNECTIVITY_RADIUS * CONNECTIVITY_RADIUS)
    valid = (row_ids != (col_base + cols)) & (d2 <= r2)
    val = jnp.where(valid, d2, jnp.inf)
    idx = cols

    val, idx = _sort_chunks_128(val, idx)
    while val.shape[0] > 128:
        val, idx = _merge_halve(val, idx)

    out_ref[...] = jnp.where(val < jnp.inf, idx + col_base, -1)


def kernel(particle_locations, num_particles_per_example):
    del num_particles_per_example
    senders_t = pl.pallas_call(
        _topk_body,
        grid=(N // R,),
        in_specs=[
            pl.BlockSpec((R, 2), lambda i: (i, 0)),
            pl.BlockSpec((HALF, 2), lambda i: ((i * R) // HALF, 0)),
        ],
        out_specs=pl.BlockSpec((MAX_NUM_NEIGHBORS, R), lambda i: (0, i)),
        out_shape=jax.ShapeDtypeStruct((MAX_NUM_NEIGHBORS, N), jnp.int32),
    )(particle_locations, particle_locations)
    senders = senders_t.T
    receivers = jnp.where(senders >= 0, jnp.arange(N, dtype=jnp.int32)[:, None], -1)
    return receivers.reshape(-1), senders.reshape(-1)


# speedup vs baseline: 4.4246x; 2.2250x over previous
import jax
import jax.numpy as jnp
from jax.experimental import pallas as pl

CONNECTIVITY_RADIUS = 0.5
MAX_NUM_NEIGHBORS = 128
N = 8192
HALF = N // 2
R = 512


def _lex_le(av, ai, bv, bi):
    return (av < bv) | ((av == bv) & (ai < bi))


def _ce(av, ai, bv, bi, asc):
    a_first = _lex_le(av, ai, bv, bi)
    take_a = a_first == asc
    fv = jnp.where(take_a, av, bv)
    sv = jnp.where(take_a, bv, av)
    fi = jnp.where(take_a, ai, bi)
    si = jnp.where(take_a, bi, ai)
    return fv, fi, sv, si


def _sort_chunks_128(val, idx):
    n = val.shape[0]
    k = 2
    while k <= 128:
        j = k // 2
        while j >= 1:
            g = n // (2 * j)
            v = val.reshape(g, 2, j, R)
            x = idx.reshape(g, 2, j, R)
            gids = jax.lax.broadcasted_iota(jnp.int32, (g, j, R), 0)
            asc = (((gids * (2 * j)) % 128) & k) == 0
            fv, fi, sv, si = _ce(v[:, 0], x[:, 0], v[:, 1], x[:, 1], asc)
            val = jnp.concatenate([fv[:, None], sv[:, None]], axis=1)
            val = val.reshape(n, R)
            idx = jnp.concatenate([fi[:, None], si[:, None]], axis=1)
            idx = idx.reshape(n, R)
            j //= 2
        k *= 2
    return val, idx


def _rev128(x):
    m = x.shape[0]
    j = 64
    while j >= 1:
        y = x.reshape(m, 128 // (2 * j), 2, j, R)
        x = jnp.concatenate([y[:, :, 1:2], y[:, :, 0:1]], axis=2)
        x = x.reshape(m, 128, R)
        j //= 2
    return x


def _merge_halve(val, idx):
    m = val.shape[0] // 256
    v = val.reshape(m, 2, 128, R)
    x = idx.reshape(m, 2, 128, R)
    av, ai = v[:, 0], x[:, 0]
    bv, bi = _rev128(v[:, 1]), _rev128(x[:, 1])
    a_first = _lex_le(av, ai, bv, bi)
    val = jnp.where(a_first, av, bv)
    idx = jnp.where(a_first, ai, bi)
    j = 64
    while j >= 1:
        g = 128 // (2 * j)
        v = val.reshape(m, g, 2, j, R)
        x = idx.reshape(m, g, 2, j, R)
        fv, fi, sv, si = _ce(v[:, :, 0], x[:, :, 0], v[:, :, 1], x[:, :, 1], True)
        val = jnp.concatenate([fv[:, :, None], sv[:, :, None]], axis=2)
        val = val.reshape(m, 128, R)
        idx = jnp.concatenate([fi[:, :, None], si[:, :, None]], axis=2)
        idx = idx.reshape(m, 128, R)
        j //= 2
    return val.reshape(m * 128, R), idx.reshape(m * 128, R)


def _topk_body(p_rows_ref, p_cols_ref, out_ref):
    pr = p_rows_ref[...]
    pc = p_cols_ref[...]
    sq_r = jnp.sum(pr * pr, axis=1)
    sq_c = jnp.sum(pc * pc, axis=1)
    cross = jax.lax.dot_general(pc, pr, (((1,), (1,)), ((), ())),
                                preferred_element_type=jnp.float32)
    d2 = sq_c[:, None] + sq_r[None, :] - 2.0 * cross
    d2 = jnp.maximum(d2, 0.0)

    i = pl.program_id(0)
    row_ids = i * R + jax.lax.broadcasted_iota(jnp.int32, d2.shape, 1)
    col_base = (i * R) // HALF * HALF
    cols = jax.lax.broadcasted_iota(jnp.int32, d2.shape, 0)
    r2 = jnp.float32(CONNECTIVITY_RADIUS * CONNECTIVITY_RADIUS)
    valid = (row_ids != (col_base + cols)) & (d2 <= r2)
    val = jnp.where(valid, d2, jnp.inf)
    idx = cols

    val, idx = _sort_chunks_128(val, idx)
    while val.shape[0] > 128:
        val, idx = _merge_halve(val, idx)

    out_ref[...] = jnp.where(val < jnp.inf, idx + col_base, -1)


def kernel(particle_locations, num_particles_per_example):
    del num_particles_per_example
    senders_t = pl.pallas_call(
        _topk_body,
        grid=(N // R,),
        in_specs=[
            pl.BlockSpec((R, 2), lambda i: (i, 0)),
            pl.BlockSpec((HALF, 2), lambda i: ((i * R) // HALF, 0)),
        ],
        out_specs=pl.BlockSpec((MAX_NUM_NEIGHBORS, R), lambda i: (0, i)),
        out_shape=jax.ShapeDtypeStruct((MAX_NUM_NEIGHBORS, N), jnp.int32),
    )(particle_locations, particle_locations)
    senders = senders_t.T
    receivers = jnp.where(senders >= 0, jnp.arange(N, dtype=jnp.int32)[:, None], -1)
    return receivers.reshape(-1), senders.reshape(-1)


# speedup vs baseline: 8.5405x; 1.9302x over previous
import jax
import jax.numpy as jnp
from jax.experimental import pallas as pl

CONNECTIVITY_RADIUS = 0.5
MAX_NUM_NEIGHBORS = 128
N = 8192
HALF = N // 2
R = 1024


def _lex_le(av, ai, bv, bi):
    return (av < bv) | ((av == bv) & (ai < bi))


def _ce(av, ai, bv, bi, asc):
    a_first = _lex_le(av, ai, bv, bi)
    take_a = a_first == asc
    fv = jnp.where(take_a, av, bv)
    sv = jnp.where(take_a, bv, av)
    fi = jnp.where(take_a, ai, bi)
    si = jnp.where(take_a, bi, ai)
    return fv, fi, sv, si


def _sort_chunks_128(val, idx):
    n = val.shape[0]
    k = 2
    while k <= 128:
        j = k // 2
        while j >= 1:
            g = n // (2 * j)
            v = val.reshape(g, 2, j, R)
            x = idx.reshape(g, 2, j, R)
            gids = jax.lax.broadcasted_iota(jnp.int32, (g, j, R), 0)
            asc = (((gids * (2 * j)) % 128) & k) == 0
            fv, fi, sv, si = _ce(v[:, 0], x[:, 0], v[:, 1], x[:, 1], asc)
            val = jnp.concatenate([fv[:, None], sv[:, None]], axis=1)
            val = val.reshape(n, R)
            idx = jnp.concatenate([fi[:, None], si[:, None]], axis=1)
            idx = idx.reshape(n, R)
            j //= 2
        k *= 2
    return val, idx


def _rev128(x):
    m = x.shape[0]
    j = 64
    while j >= 1:
        y = x.reshape(m, 128 // (2 * j), 2, j, R)
        x = jnp.concatenate([y[:, :, 1:2], y[:, :, 0:1]], axis=2)
        x = x.reshape(m, 128, R)
        j //= 2
    return x


def _merge_halve(val, idx):
    m = val.shape[0] // 256
    v = val.reshape(m, 2, 128, R)
    x = idx.reshape(m, 2, 128, R)
    av, ai = v[:, 0], x[:, 0]
    bv, bi = _rev128(v[:, 1]), _rev128(x[:, 1])
    a_first = _lex_le(av, ai, bv, bi)
    val = jnp.where(a_first, av, bv)
    idx = jnp.where(a_first, ai, bi)
    j = 64
    while j >= 1:
        g = 128 // (2 * j)
        v = val.reshape(m, g, 2, j, R)
        x = idx.reshape(m, g, 2, j, R)
        fv, fi, sv, si = _ce(v[:, :, 0], x[:, :, 0], v[:, :, 1], x[:, :, 1], True)
        val = jnp.concatenate([fv[:, :, None], sv[:, :, None]], axis=2)
        val = val.reshape(m, 128, R)
        idx = jnp.concatenate([fi[:, :, None], si[:, :, None]], axis=2)
        idx = idx.reshape(m, 128, R)
        j //= 2
    return val.reshape(m * 128, R), idx.reshape(m * 128, R)


def _topk_body(p_rows_ref, p_cols_ref, out_ref):
    pr = p_rows_ref[...]
    pc = p_cols_ref[...]
    sq_r = jnp.sum(pr * pr, axis=1)
    sq_c = jnp.sum(pc * pc, axis=1)
    cross = jax.lax.dot_general(pc, pr, (((1,), (1,)), ((), ())),
                                preferred_element_type=jnp.float32)
    d2 = sq_c[:, None] + sq_r[None, :] - 2.0 * cross
    d2 = jnp.maximum(d2, 0.0)

    i = pl.program_id(0)
    row_ids = i * R + jax.lax.broadcasted_iota(jnp.int32, d2.shape, 1)
    col_base = (i * R) // HALF * HALF
    cols = jax.lax.broadcasted_iota(jnp.int32, d2.shape, 0)
    r2 = jnp.float32(CONNECTIVITY_RADIUS * CONNECTIVITY_RADIUS)
    valid = (row_ids != (col_base + cols)) & (d2 <= r2)
    val = jnp.where(valid, d2, jnp.inf)
    idx = cols

    val, idx = _sort_chunks_128(val, idx)
    while val.shape[0] > 128:
        val, idx = _merge_halve(val, idx)

    out_ref[...] = jnp.where(val < jnp.inf, idx + col_base, -1)


def kernel(particle_locations, num_particles_per_example):
    del num_particles_per_example
    senders_t = pl.pallas_call(
        _topk_body,
        grid=(N // R,),
        in_specs=[
            pl.BlockSpec((R, 2), lambda i: (i, 0)),
            pl.BlockSpec((HALF, 2), lambda i: ((i * R) // HALF, 0)),
        ],
        out_specs=pl.BlockSpec((MAX_NUM_NEIGHBORS, R), lambda i: (0, i)),
        out_shape=jax.ShapeDtypeStruct((MAX_NUM_NEIGHBORS, N), jnp.int32),
    )(particle_locations, particle_locations)
    senders = senders_t.T
    receivers = jnp.where(senders >= 0, jnp.arange(N, dtype=jnp.int32)[:, None], -1)
    return receivers.reshape(-1), senders.reshape(-1)
